# Initial kernel scaffold; baseline (speedup 1.0000x reference)
import jax
import jax.numpy as jnp
from jax.experimental import pallas as pl


def kernel(dst_in, indices_in, src_in, dim_i):
    raise NotImplementedError("write your pallas kernel here")


# speedup vs baseline: 1.0280x; 1.0280x over previous
import jax
import jax.numpy as jnp
from jax.experimental import pallas as pl
from jax.experimental.pallas import tpu as pltpu


def _update_body(offs_ref, dst_ref, src_ref, out_ref):
    b = pl.program_id(0)
    out_ref[...] = dst_ref[...]
    off = offs_ref[b]
    q = src_ref.shape[2]
    out_ref[0, 0, pl.ds(off, q), :] = src_ref[0, 0, :, :]


def kernel(dst_in, indices_in, src_in, dim_i):
    B, H, S, D = dst_in.shape
    Q = src_in.shape[2]
    offs = (indices_in + (jnp.asarray(dim_i) - 2)).astype(jnp.int32)

    grid_spec = pltpu.PrefetchScalarGridSpec(
        num_scalar_prefetch=1,
        grid=(B, H),
        in_specs=[
            pl.BlockSpec((1, 1, S, D), lambda b, h, offs: (b, h, 0, 0)),
            pl.BlockSpec((1, 1, Q, D), lambda b, h, offs: (b, h, 0, 0)),
        ],
        out_specs=pl.BlockSpec((1, 1, S, D), lambda b, h, offs: (b, h, 0, 0)),
    )
    return pl.pallas_call(
        _update_body,
        grid_spec=grid_spec,
        out_shape=jax.ShapeDtypeStruct(dst_in.shape, dst_in.dtype),
    )(offs, dst_in, src_in)


# speedup vs baseline: 1.5673x; 1.5246x over previous
import jax
import jax.numpy as jnp
from jax.experimental import pallas as pl
from jax.experimental.pallas import tpu as pltpu


def _update_body(offs_ref, dst_ref, src_ref, out_ref):
    b = pl.program_id(0)
    out_ref[...] = dst_ref[...]
    off = offs_ref[b]
    q = src_ref.shape[2]
    for j in range(src_ref.shape[1]):
        out_ref[0, j, pl.ds(off, q), :] = src_ref[0, j, :, :]


def kernel(dst_in, indices_in, src_in, dim_i):
    B, H, S, D = dst_in.shape
    Q = src_in.shape[2]
    offs = (indices_in + (jnp.asarray(dim_i) - 2)).astype(jnp.int32)

    HB = 4
    grid_spec = pltpu.PrefetchScalarGridSpec(
        num_scalar_prefetch=1,
        grid=(B, H // HB),
        in_specs=[
            pl.BlockSpec((1, HB, S, D), lambda b, h, offs: (b, h, 0, 0)),
            pl.BlockSpec((1, HB, Q, D), lambda b, h, offs: (b, h, 0, 0)),
        ],
        out_specs=pl.BlockSpec((1, HB, S, D), lambda b, h, offs: (b, h, 0, 0)),
    )
    return pl.pallas_call(
        _update_body,
        grid_spec=grid_spec,
        out_shape=jax.ShapeDtypeStruct(dst_in.shape, dst_in.dtype),
    )(offs, dst_in, src_in)


# speedup vs baseline: 1.5775x; 1.0065x over previous
import jax
import jax.numpy as jnp
from jax.experimental import pallas as pl
from jax.experimental.pallas import tpu as pltpu


def _update_body(offs_ref, dst_ref, src_ref, out_ref):
    b = pl.program_id(0)
    out_ref[...] = dst_ref[...]
    off = offs_ref[b]
    q = src_ref.shape[2]
    for j in range(src_ref.shape[1]):
        out_ref[0, j, pl.ds(off, q), :] = src_ref[0, j, :, :]


def kernel(dst_in, indices_in, src_in, dim_i):
    B, H, S, D = dst_in.shape
    Q = src_in.shape[2]
    offs = (indices_in + (jnp.asarray(dim_i) - 2)).astype(jnp.int32)

    HB = 8
    grid_spec = pltpu.PrefetchScalarGridSpec(
        num_scalar_prefetch=1,
        grid=(B, H // HB),
        in_specs=[
            pl.BlockSpec((1, HB, S, D), lambda b, h, offs: (b, h, 0, 0)),
            pl.BlockSpec((1, HB, Q, D), lambda b, h, offs: (b, h, 0, 0)),
        ],
        out_specs=pl.BlockSpec((1, HB, S, D), lambda b, h, offs: (b, h, 0, 0)),
    )
    return pl.pallas_call(
        _update_body,
        grid_spec=grid_spec,
        out_shape=jax.ShapeDtypeStruct(dst_in.shape, dst_in.dtype),
    )(offs, dst_in, src_in)
